# baseline (speedup 1.0000x reference)
import functools

import jax
import jax.numpy as jnp
from jax import lax
from jax.experimental import pallas as pl
from jax.experimental.pallas import tpu as pltpu
from jax.experimental.pallas import tpu_sc as plsc

_VOCAB = 1000000
_SEQ_LEN = 200
_EMBED_DIM = 32
_BATCH = 4096

_NC = 2
_NS = 16
_NW = _NC * _NS

_BBLK = _BATCH // _NW
_S_PER_CHUNK = 4
_CHUNK_ROWS = _S_PER_CHUNK * _BBLK
_N_CHUNKS = _SEQ_LEN // _S_PER_CHUNK

_ST = _SEQ_LEN // 8
_BT = _BATCH // 128

_EG = _EMBED_DIM // 8
_TILE = 8 * 128
_PITCH = 129

_HALF = 16


def _emb_body(idx_hbm, tok_hbm, pos_hbm, out_hbm,
              idx0, idx1, g0, g1, t0, t1, pos_v,
              gsem0, gsem1, osem0, osem1):
    idxs = (idx0, idx1)
    gs = (g0, g1)
    tiles = (t0, t1)
    gsems = (gsem0, gsem1)
    osems = (osem0, osem1)

    wid = lax.axis_index("s") * _NC + lax.axis_index("c")

    pltpu.sync_copy(pos_hbm, pos_v)

    lane = lax.iota(jnp.int32, 16)
    gv0 = lane >> 3
    gv1 = gv0 + 2
    e8v = lane & 7

    def idx_and_gathers(c, b):
        pltpu.sync_copy(
            idx_hbm.at[c // 2, wid, pl.ds((c % 2) * _S_PER_CHUNK, _S_PER_CHUNK)],
            idxs[b],
        )
        for d in gather_descs(c, b):
            d.start()

    def gather_descs(c, b):
        return [
            pltpu.make_async_copy(
                tok_hbm.at[idxs[b].at[j]],
                gs[b].at[pl.ds(j * _BBLK, _BBLK)],
                gsems[b],
            )
            for j in range(_S_PER_CHUNK)
        ]

    def drain_gathers(c, b):
        for d in gather_descs(c, b):
            d.wait()

    def transpose_add(c, b):
        for j in range(_S_PER_CHUNK):
            s = c * _S_PER_CHUNK + j
            pv0 = pos_v[s, pl.ds(0, _HALF)]
            pv1 = pos_v[s, pl.ds(_HALF, _HALF)]

            @plsc.parallel_loop(0, _BBLK, unroll=4)
            def _(bl):
                r = j * _BBLK + bl
                wv = lax.broadcast_in_dim(bl, (16,), ())
                v0 = gs[b][r, pl.ds(0, _HALF)] + pv0
                v1 = gs[b][r, pl.ds(_HALF, _HALF)] + pv1
                plsc.store_scatter(tiles[b].at[j], [gv0, e8v, wv], v0)
                plsc.store_scatter(tiles[b].at[j], [gv1, e8v, wv], v1)

    def out_descs(c, b):
        ds_ = []
        for j in range(_S_PER_CHUNK):
            s = c * _S_PER_CHUNK + j
            ds_.append(
                pltpu.make_async_copy(
                    tiles[b].at[j, slice(None), slice(None), pl.ds(0, 128)],
                    out_hbm.at[s, slice(None), wid],
                    osems[b],
                )
            )
        return ds_

    idx_and_gathers(0, 0)
    idx_and_gathers(1, 1)

    for b in range(2):
        drain_gathers(b, b)
        transpose_add(b, b)
        for d in out_descs(b, b):
            d.start()
        idx_and_gathers(b + 2, b)

    @pl.loop(2, _N_CHUNKS - 2, step=2)
    def _(sc):
        for b in range(2):
            c = sc + b
            drain_gathers(c, b)
            for d in out_descs(c - 2, b):
                d.wait()
            transpose_add(c, b)
            for d in out_descs(c, b):
                d.start()
            idx_and_gathers(c + 2, b)

    for b in range(2):
        c = _N_CHUNKS - 2 + b
        drain_gathers(c, b)
        for d in out_descs(c - 2, b):
            d.wait()
        transpose_add(c, b)
        for d in out_descs(c, b):
            d.start()

    for b in range(2):
        for d in out_descs(_N_CHUNKS - 2 + b, b):
            d.wait()


_emb = functools.partial(
    pl.kernel,
    out_type=jax.ShapeDtypeStruct((_SEQ_LEN, _EG, _BT, 8, 128), jnp.float32),
    mesh=plsc.VectorSubcoreMesh(core_axis_name="c", subcore_axis_name="s"),
    scratch_types=[
        pltpu.VMEM((_S_PER_CHUNK, _BBLK), jnp.int32),
        pltpu.VMEM((_S_PER_CHUNK, _BBLK), jnp.int32),
        pltpu.VMEM((_CHUNK_ROWS, _EMBED_DIM), jnp.float32),
        pltpu.VMEM((_CHUNK_ROWS, _EMBED_DIM), jnp.float32),
        pltpu.VMEM((_S_PER_CHUNK, _EG, 8, _PITCH), jnp.float32),
        pltpu.VMEM((_S_PER_CHUNK, _EG, 8, _PITCH), jnp.float32),
        pltpu.VMEM((_SEQ_LEN, _EMBED_DIM), jnp.float32),
        pltpu.SemaphoreType.DMA,
        pltpu.SemaphoreType.DMA,
        pltpu.SemaphoreType.DMA,
        pltpu.SemaphoreType.DMA,
    ],
    compiler_params=pltpu.CompilerParams(
        use_tc_tiling_on_sc=False, needs_layout_passes=False
    ),
)(_emb_body)


_TCB = 2048


def _repack_body(t_ref, o_ref):
    x = t_ref[...]
    x = x.reshape(_EMBED_DIM, _TCB // 4, 4)
    x = jnp.transpose(x, (1, 2, 0))
    o_ref[...] = x.reshape(_TCB // 4, 128)


_repack = pl.pallas_call(
    _repack_body,
    grid=((_VOCAB + _TCB - 1) // _TCB,),
    in_specs=[pl.BlockSpec((_EMBED_DIM, _TCB), lambda i: (0, i))],
    out_specs=pl.BlockSpec((_TCB // 4, 128), lambda i: (i, 0)),
    out_shape=jax.ShapeDtypeStruct((_VOCAB // 4, 128), jnp.float32),
)


@jax.jit
def kernel(inputs, token_table, position_table):
    idx4d = (
        inputs.astype(jnp.int32).T
        .reshape(_ST, 8, _BT, 128)
        .transpose(0, 2, 1, 3)
    )
    rm_table = _repack(token_table.T).reshape(_VOCAB, _EMBED_DIM)
    flat = _emb(idx4d, rm_table, position_table)
    out = (
        flat.reshape(_SEQ_LEN, _EG, _BT, 8, 128)
        .transpose(2, 4, 0, 1, 3)
        .reshape(_BATCH, _SEQ_LEN, _EMBED_DIM)
    )
    return out


# speedup vs baseline: 4.2119x; 4.2119x over previous
import functools

import jax
import jax.numpy as jnp
from jax import lax
from jax.experimental import pallas as pl
from jax.experimental.pallas import tpu as pltpu
from jax.experimental.pallas import tpu_sc as plsc

_VOCAB = 1000000
_SEQ_LEN = 200
_EMBED_DIM = 32
_BATCH = 4096

_NC = 2
_NS = 16
_NW = _NC * _NS

_BBLK = _BATCH // _NW
_S_PER_CHUNK = 4
_CHUNK_ROWS = _S_PER_CHUNK * _BBLK
_N_CHUNKS = _SEQ_LEN // _S_PER_CHUNK

_ST = _SEQ_LEN // 8
_BT = _BATCH // 128

_EG = _EMBED_DIM // 8
_TILE = 8 * 128
_PITCH = 129

_HALF = 16


def _emb_body(idx_hbm, tok_hbm, pos_hbm, out_hbm,
              idx0, idx1, g0, g1, t0, t1, pos_v,
              gsem0, gsem1, osem0, osem1):
    idxs = (idx0, idx1)
    gs = (g0, g1)
    tiles = (t0, t1)
    gsems = (gsem0, gsem1)
    osems = (osem0, osem1)

    wid = lax.axis_index("s") * _NC + lax.axis_index("c")

    pltpu.sync_copy(pos_hbm, pos_v)

    lane = lax.iota(jnp.int32, 16)
    gv0 = lane >> 3
    gv1 = gv0 + 2
    e8v = lane & 7

    def idx_and_gathers(c, b):
        pltpu.sync_copy(
            idx_hbm.at[c // 2, wid, pl.ds((c % 2) * _S_PER_CHUNK, _S_PER_CHUNK)],
            idxs[b],
        )
        for d in gather_descs(c, b):
            d.start()

    def gather_descs(c, b):
        return [
            pltpu.make_async_copy(
                tok_hbm.at[idxs[b].at[j]],
                gs[b].at[pl.ds(j * _BBLK, _BBLK)],
                gsems[b],
            )
            for j in range(_S_PER_CHUNK)
        ]

    def drain_gathers(c, b):
        for d in gather_descs(c, b):
            d.wait()

    def transpose_add(c, b):
        for j in range(_S_PER_CHUNK):
            s = c * _S_PER_CHUNK + j
            pv0 = pos_v[s, pl.ds(0, _HALF)]
            pv1 = pos_v[s, pl.ds(_HALF, _HALF)]

            @plsc.parallel_loop(0, _BBLK, unroll=4)
            def _(bl):
                r = j * _BBLK + bl
                wv = lax.broadcast_in_dim(bl, (16,), ())
                v0 = gs[b][r, pl.ds(0, _HALF)] + pv0
                v1 = gs[b][r, pl.ds(_HALF, _HALF)] + pv1
                plsc.store_scatter(tiles[b].at[j], [gv0, e8v, wv], v0)
                plsc.store_scatter(tiles[b].at[j], [gv1, e8v, wv], v1)

    def out_descs(c, b):
        ds_ = []
        for j in range(_S_PER_CHUNK):
            s = c * _S_PER_CHUNK + j
            ds_.append(
                pltpu.make_async_copy(
                    tiles[b].at[j, slice(None), slice(None), pl.ds(0, 128)],
                    out_hbm.at[s, slice(None), wid],
                    osems[b],
                )
            )
        return ds_

    idx_and_gathers(0, 0)
    idx_and_gathers(1, 1)

    for b in range(2):
        drain_gathers(b, b)
        transpose_add(b, b)
        for d in out_descs(b, b):
            d.start()
        idx_and_gathers(b + 2, b)

    @pl.loop(2, _N_CHUNKS - 2, step=2)
    def _(sc):
        for b in range(2):
            c = sc + b
            drain_gathers(c, b)
            for d in out_descs(c - 2, b):
                d.wait()
            transpose_add(c, b)
            for d in out_descs(c, b):
                d.start()
            idx_and_gathers(c + 2, b)

    for b in range(2):
        c = _N_CHUNKS - 2 + b
        drain_gathers(c, b)
        for d in out_descs(c - 2, b):
            d.wait()
        transpose_add(c, b)
        for d in out_descs(c, b):
            d.start()

    for b in range(2):
        for d in out_descs(_N_CHUNKS - 2 + b, b):
            d.wait()


_emb = functools.partial(
    pl.kernel,
    out_type=jax.ShapeDtypeStruct((_SEQ_LEN, _EG, _BT, 8, 128), jnp.float32),
    mesh=plsc.VectorSubcoreMesh(core_axis_name="c", subcore_axis_name="s"),
    scratch_types=[
        pltpu.VMEM((_S_PER_CHUNK, _BBLK), jnp.int32),
        pltpu.VMEM((_S_PER_CHUNK, _BBLK), jnp.int32),
        pltpu.VMEM((_CHUNK_ROWS, _EMBED_DIM), jnp.float32),
        pltpu.VMEM((_CHUNK_ROWS, _EMBED_DIM), jnp.float32),
        pltpu.VMEM((_S_PER_CHUNK, _EG, 8, _PITCH), jnp.float32),
        pltpu.VMEM((_S_PER_CHUNK, _EG, 8, _PITCH), jnp.float32),
        pltpu.VMEM((_SEQ_LEN, _EMBED_DIM), jnp.float32),
        pltpu.SemaphoreType.DMA,
        pltpu.SemaphoreType.DMA,
        pltpu.SemaphoreType.DMA,
        pltpu.SemaphoreType.DMA,
    ],
    compiler_params=pltpu.CompilerParams(
        use_tc_tiling_on_sc=False, needs_layout_passes=False
    ),
)(_emb_body)


@jax.jit
def kernel(inputs, token_table, position_table):
    idx4d = (
        inputs.astype(jnp.int32).T
        .reshape(_ST, 8, _BT, 128)
        .transpose(0, 2, 1, 3)
    )
    flat = _emb(idx4d, token_table, position_table)
    out = (
        flat.reshape(_SEQ_LEN, _EG, _BT, 8, 128)
        .transpose(2, 4, 0, 1, 3)
        .reshape(_BATCH, _SEQ_LEN, _EMBED_DIM)
    )
    return out


# speedup vs baseline: 4.6025x; 1.0927x over previous
import functools

import jax
import jax.numpy as jnp
from jax import lax
from jax.experimental import pallas as pl
from jax.experimental.pallas import tpu as pltpu
from jax.experimental.pallas import tpu_sc as plsc

_VOCAB = 1000000
_SEQ_LEN = 200
_EMBED_DIM = 32
_BATCH = 4096

_NC = 2
_NS = 16
_NW = _NC * _NS

_BBLK = _BATCH // _NW
_S_PER_CHUNK = 4
_CHUNK_ROWS = _S_PER_CHUNK * _BBLK
_N_CHUNKS = _SEQ_LEN // _S_PER_CHUNK

_ST = _SEQ_LEN // 8
_BT = _BATCH // 128

_EG = _EMBED_DIM // 8
_TILE = 8 * 128
_PITCH = 129

_HALF = 16


def _emb_body(idx_hbm, tok_hbm, pos_hbm, out_hbm,
              idx0, idx1, g0, g1, t0, t1, pos_v,
              gsem0, gsem1, osem0, osem1):
    idxs = (idx0, idx1)
    gs = (g0, g1)
    tiles = (t0, t1)
    gsems = (gsem0, gsem1)
    osems = (osem0, osem1)

    wid = lax.axis_index("s") * _NC + lax.axis_index("c")

    pltpu.sync_copy(pos_hbm, pos_v)

    lane = lax.iota(jnp.int32, 16)
    gv0 = lane >> 3
    gv1 = gv0 + 2
    e8v = lane & 7

    def idx_and_gathers(c, b):
        pltpu.sync_copy(
            idx_hbm.at[c // 2, wid, pl.ds((c % 2) * _S_PER_CHUNK, _S_PER_CHUNK)],
            idxs[b],
        )
        for d in gather_descs(c, b):
            d.start()

    def gather_descs(c, b):
        return [
            pltpu.make_async_copy(
                tok_hbm.at[idxs[b].at[j]],
                gs[b].at[pl.ds(j * _BBLK, _BBLK)],
                gsems[b],
            )
            for j in range(_S_PER_CHUNK)
        ]

    def drain_gathers(c, b):
        for d in gather_descs(c, b):
            d.wait()

    def transpose_add(c, b):
        for j in range(_S_PER_CHUNK):
            s = c * _S_PER_CHUNK + j
            pv0 = pos_v[s, pl.ds(0, _HALF)]
            pv1 = pos_v[s, pl.ds(_HALF, _HALF)]

            @plsc.parallel_loop(0, _BBLK, unroll=4)
            def _(bl):
                r = j * _BBLK + bl
                wv = lax.broadcast_in_dim(bl, (16,), ())
                v0 = gs[b][r, pl.ds(0, _HALF)] + pv0
                v1 = gs[b][r, pl.ds(_HALF, _HALF)] + pv1
                plsc.store_scatter(tiles[b].at[j], [gv0, e8v, wv], v0)
                plsc.store_scatter(tiles[b].at[j], [gv1, e8v, wv], v1)

    def out_descs(c, b):
        ds_ = []
        for j in range(_S_PER_CHUNK):
            s = c * _S_PER_CHUNK + j
            ds_.append(
                pltpu.make_async_copy(
                    tiles[b].at[j, slice(None), slice(None), pl.ds(0, 128)],
                    out_hbm.at[s, slice(None), wid],
                    osems[b],
                )
            )
        return ds_

    idx_and_gathers(0, 0)
    idx_and_gathers(1, 1)

    for b in range(2):
        drain_gathers(b, b)
        transpose_add(b, b)
        for d in out_descs(b, b):
            d.start()
        idx_and_gathers(b + 2, b)

    @pl.loop(2, _N_CHUNKS - 2, step=2)
    def _(sc):
        for b in range(2):
            c = sc + b
            drain_gathers(c, b)
            for d in out_descs(c - 2, b):
                d.wait()
            transpose_add(c, b)
            for d in out_descs(c, b):
                d.start()
            idx_and_gathers(c + 2, b)

    for b in range(2):
        c = _N_CHUNKS - 2 + b
        drain_gathers(c, b)
        for d in out_descs(c - 2, b):
            d.wait()
        transpose_add(c, b)
        for d in out_descs(c, b):
            d.start()

    for b in range(2):
        for d in out_descs(_N_CHUNKS - 2 + b, b):
            d.wait()


_emb = functools.partial(
    pl.kernel,
    out_type=jax.ShapeDtypeStruct((_SEQ_LEN, _EG, _BT, 8, 128), jnp.float32),
    mesh=plsc.VectorSubcoreMesh(core_axis_name="c", subcore_axis_name="s"),
    scratch_types=[
        pltpu.VMEM((_S_PER_CHUNK, _BBLK), jnp.int32),
        pltpu.VMEM((_S_PER_CHUNK, _BBLK), jnp.int32),
        pltpu.VMEM((_CHUNK_ROWS, _EMBED_DIM), jnp.float32),
        pltpu.VMEM((_CHUNK_ROWS, _EMBED_DIM), jnp.float32),
        pltpu.VMEM((_S_PER_CHUNK, _EG, 8, _PITCH), jnp.float32),
        pltpu.VMEM((_S_PER_CHUNK, _EG, 8, _PITCH), jnp.float32),
        pltpu.VMEM((_SEQ_LEN, _EMBED_DIM), jnp.float32),
        pltpu.SemaphoreType.DMA,
        pltpu.SemaphoreType.DMA,
        pltpu.SemaphoreType.DMA,
        pltpu.SemaphoreType.DMA,
    ],
    compiler_params=pltpu.CompilerParams(
        use_tc_tiling_on_sc=False, needs_layout_passes=False
    ),
)(_emb_body)


_TCB = 2048


def _repack_body(t_ref, o_ref):
    o_ref[:, 0:_EMBED_DIM] = t_ref[...].T


_repack = pl.pallas_call(
    _repack_body,
    grid=((_VOCAB + _TCB - 1) // _TCB,),
    in_specs=[pl.BlockSpec((_EMBED_DIM, _TCB), lambda i: (0, i))],
    out_specs=pl.BlockSpec((_TCB, 128), lambda i: (i, 0)),
    out_shape=jax.ShapeDtypeStruct((_VOCAB, 128), jnp.float32),
)


@jax.jit
def kernel(inputs, token_table, position_table):
    idx4d = (
        inputs.astype(jnp.int32).T
        .reshape(_ST, 8, _BT, 128)
        .transpose(0, 2, 1, 3)
    ) * 4
    rm_table = _repack(token_table.T).reshape(4 * _VOCAB, _EMBED_DIM)
    flat = _emb(idx4d, rm_table, position_table)
    out = (
        flat.reshape(_SEQ_LEN, _EG, _BT, 8, 128)
        .transpose(2, 4, 0, 1, 3)
        .reshape(_BATCH, _SEQ_LEN, _EMBED_DIM)
    )
    return out


# speedup vs baseline: 7.0956x; 1.5417x over previous
import functools

import jax
import jax.numpy as jnp
from jax import lax
from jax.experimental import pallas as pl
from jax.experimental.pallas import tpu as pltpu
from jax.experimental.pallas import tpu_sc as plsc

_VOCAB = 1000000
_SEQ_LEN = 200
_EMBED_DIM = 32
_BATCH = 4096

_NC = 2
_NS = 16
_NW = _NC * _NS

_BBLK = _BATCH // _NW
_S_PER_CHUNK = 4
_CHUNK_ROWS = _S_PER_CHUNK * _BBLK
_N_CHUNKS = _SEQ_LEN // _S_PER_CHUNK

_ST = _SEQ_LEN // 8
_BT = _BATCH // 128

_EG = _EMBED_DIM // 8
_TILE = 8 * 128
_PITCH = 129

_HALF = 16


def _emb_body(idx_hbm, tok_hbm, pos_hbm, out_hbm,
              idx0, idx1, g0, g1, t0, t1, pos_v,
              gsem0, gsem1, osem0, osem1):
    idxs = (idx0, idx1)
    gs = (g0, g1)
    tiles = (t0, t1)
    gsems = (gsem0, gsem1)
    osems = (osem0, osem1)

    wid = lax.axis_index("s") * _NC + lax.axis_index("c")

    pltpu.sync_copy(pos_hbm, pos_v)

    lane = lax.iota(jnp.int32, 16)
    gv0 = lane >> 3
    gv1 = gv0 + 2
    e8v = lane & 7

    def idx_and_gathers(c, b):
        pltpu.sync_copy(
            idx_hbm.at[c // 2, wid, pl.ds((c % 2) * _S_PER_CHUNK, _S_PER_CHUNK)],
            idxs[b],
        )
        for d in gather_descs(c, b):
            d.start()

    def gather_descs(c, b):
        return [
            pltpu.make_async_copy(
                tok_hbm.at[idxs[b].at[j]],
                gs[b].at[pl.ds(j * _BBLK, _BBLK)],
                gsems[b],
            )
            for j in range(_S_PER_CHUNK)
        ]

    def drain_gathers(c, b):
        for d in gather_descs(c, b):
            d.wait()

    def transpose_add(c, b):
        for j in range(_S_PER_CHUNK):
            s = c * _S_PER_CHUNK + j
            pv0 = pos_v[s, pl.ds(0, _HALF)]
            pv1 = pos_v[s, pl.ds(_HALF, _HALF)]

            @plsc.parallel_loop(0, _BBLK, unroll=4)
            def _(bl):
                r = j * _BBLK + bl
                wv = lax.broadcast_in_dim(bl, (16,), ())
                v0 = gs[b][r, pl.ds(0, _HALF)] + pv0
                v1 = gs[b][r, pl.ds(_HALF, _HALF)] + pv1
                plsc.store_scatter(tiles[b].at[j], [gv0, e8v, wv], v0)
                plsc.store_scatter(tiles[b].at[j], [gv1, e8v, wv], v1)

    def out_descs(c, b):
        ds_ = []
        for j in range(_S_PER_CHUNK):
            s = c * _S_PER_CHUNK + j
            ds_.append(
                pltpu.make_async_copy(
                    tiles[b].at[j, slice(None), slice(None), pl.ds(0, 128)],
                    out_hbm.at[s, slice(None), wid],
                    osems[b],
                )
            )
        return ds_

    idx_and_gathers(0, 0)
    idx_and_gathers(1, 1)

    for b in range(2):
        drain_gathers(b, b)
        transpose_add(b, b)
        for d in out_descs(b, b):
            d.start()
        idx_and_gathers(b + 2, b)

    @pl.loop(2, _N_CHUNKS - 2, step=2)
    def _(sc):
        for b in range(2):
            c = sc + b
            drain_gathers(c, b)
            for d in out_descs(c - 2, b):
                d.wait()
            transpose_add(c, b)
            for d in out_descs(c, b):
                d.start()
            idx_and_gathers(c + 2, b)

    for b in range(2):
        c = _N_CHUNKS - 2 + b
        drain_gathers(c, b)
        for d in out_descs(c - 2, b):
            d.wait()
        transpose_add(c, b)
        for d in out_descs(c, b):
            d.start()

    for b in range(2):
        for d in out_descs(_N_CHUNKS - 2 + b, b):
            d.wait()


_emb = functools.partial(
    pl.kernel,
    out_type=jax.ShapeDtypeStruct((_SEQ_LEN, _EG, _BT, 8, 128), jnp.float32),
    mesh=plsc.VectorSubcoreMesh(core_axis_name="c", subcore_axis_name="s"),
    scratch_types=[
        pltpu.VMEM((_S_PER_CHUNK, _BBLK), jnp.int32),
        pltpu.VMEM((_S_PER_CHUNK, _BBLK), jnp.int32),
        pltpu.VMEM((_CHUNK_ROWS, _EMBED_DIM), jnp.float32),
        pltpu.VMEM((_CHUNK_ROWS, _EMBED_DIM), jnp.float32),
        pltpu.VMEM((_S_PER_CHUNK, _EG, 8, _PITCH), jnp.float32),
        pltpu.VMEM((_S_PER_CHUNK, _EG, 8, _PITCH), jnp.float32),
        pltpu.VMEM((_SEQ_LEN, _EMBED_DIM), jnp.float32),
        pltpu.SemaphoreType.DMA,
        pltpu.SemaphoreType.DMA,
        pltpu.SemaphoreType.DMA,
        pltpu.SemaphoreType.DMA,
    ],
    compiler_params=pltpu.CompilerParams(
        use_tc_tiling_on_sc=False, needs_layout_passes=False
    ),
)(_emb_body)


_TCB = 8192


def _repack_body(t_ref, o_ref):
    o_ref[:, 0:_EMBED_DIM] = t_ref[...].T


_repack = pl.pallas_call(
    _repack_body,
    grid=((_VOCAB + _TCB - 1) // _TCB,),
    in_specs=[pl.BlockSpec((_EMBED_DIM, _TCB), lambda i: (0, i))],
    out_specs=pl.BlockSpec((_TCB, 128), lambda i: (i, 0)),
    out_shape=jax.ShapeDtypeStruct((_VOCAB, 128), jnp.float32),
)


@jax.jit
def kernel(inputs, token_table, position_table):
    idx4d = (
        inputs.astype(jnp.int32).T
        .reshape(_ST, 8, _BT, 128)
        .transpose(0, 2, 1, 3)
    ) * 4
    rm_table = _repack(token_table.T).reshape(4 * _VOCAB, _EMBED_DIM)
    flat = _emb(idx4d, rm_table, position_table)
    out = (
        flat.reshape(_SEQ_LEN, _EG, _BT, 8, 128)
        .transpose(2, 4, 0, 1, 3)
        .reshape(_BATCH, _SEQ_LEN, _EMBED_DIM)
    )
    return out


# speedup vs baseline: 7.7774x; 1.0961x over previous
import functools

import jax
import jax.numpy as jnp
from jax import lax
from jax.experimental import pallas as pl
from jax.experimental.pallas import tpu as pltpu
from jax.experimental.pallas import tpu_sc as plsc

_VOCAB = 1000000
_SEQ_LEN = 200
_EMBED_DIM = 32
_BATCH = 4096

_NC = 2
_NS = 16
_NW = _NC * _NS

_BBLK = _BATCH // _NW
_S_PER_CHUNK = 4
_CHUNK_ROWS = _S_PER_CHUNK * _BBLK
_N_CHUNKS = _SEQ_LEN // _S_PER_CHUNK

_ST = _SEQ_LEN // 8
_BT = _BATCH // 128

_EG = _EMBED_DIM // 8
_TILE = 8 * 128
_PITCH = 129

_HALF = 16


def _emb_body(idx_hbm, tok_hbm, pos_hbm, out_hbm,
              idx0, idx1, g0, g1, t0, t1, pos_v,
              gsem0, gsem1, osem0, osem1):
    idxs = (idx0, idx1)
    gs = (g0, g1)
    tiles = (t0, t1)
    gsems = (gsem0, gsem1)
    osems = (osem0, osem1)

    wid = lax.axis_index("s") * _NC + lax.axis_index("c")

    pltpu.sync_copy(pos_hbm, pos_v)

    lane = lax.iota(jnp.int32, 16)
    gv0 = lane >> 3
    gv1 = gv0 + 2
    e8v = lane & 7

    def idx_and_gathers(c, b):
        pltpu.sync_copy(
            idx_hbm.at[c // 2, wid, pl.ds((c % 2) * _S_PER_CHUNK, _S_PER_CHUNK)],
            idxs[b],
        )
        for d in gather_descs(c, b):
            d.start()

    def gather_descs(c, b):
        return [
            pltpu.make_async_copy(
                tok_hbm.at[idxs[b].at[j]],
                gs[b].at[pl.ds(j * _BBLK, _BBLK)],
                gsems[b],
            )
            for j in range(_S_PER_CHUNK)
        ]

    def drain_gathers(c, b):
        for d in gather_descs(c, b):
            d.wait()

    def transpose_add(c, b):
        for j in range(_S_PER_CHUNK):
            s = c * _S_PER_CHUNK + j
            pv0 = pos_v[s, pl.ds(0, _HALF)]
            pv1 = pos_v[s, pl.ds(_HALF, _HALF)]

            @plsc.parallel_loop(0, _BBLK, unroll=4)
            def _(bl):
                r = j * _BBLK + bl
                wv = lax.broadcast_in_dim(bl, (16,), ())
                v0 = gs[b][r, pl.ds(0, _HALF)] + pv0
                v1 = gs[b][r, pl.ds(_HALF, _HALF)] + pv1
                plsc.store_scatter(tiles[b].at[j], [gv0, e8v, wv], v0)
                plsc.store_scatter(tiles[b].at[j], [gv1, e8v, wv], v1)

    def out_descs(c, b):
        ds_ = []
        for j in range(_S_PER_CHUNK):
            s = c * _S_PER_CHUNK + j
            ds_.append(
                pltpu.make_async_copy(
                    tiles[b].at[j, slice(None), slice(None), pl.ds(0, 128)],
                    out_hbm.at[s, slice(None), wid],
                    osems[b],
                )
            )
        return ds_

    idx_and_gathers(0, 0)
    idx_and_gathers(1, 1)

    for b in range(2):
        drain_gathers(b, b)
        transpose_add(b, b)
        for d in out_descs(b, b):
            d.start()
        idx_and_gathers(b + 2, b)

    @pl.loop(2, _N_CHUNKS - 2, step=2)
    def _(sc):
        for b in range(2):
            c = sc + b
            drain_gathers(c, b)
            for d in out_descs(c - 2, b):
                d.wait()
            transpose_add(c, b)
            for d in out_descs(c, b):
                d.start()
            idx_and_gathers(c + 2, b)

    for b in range(2):
        c = _N_CHUNKS - 2 + b
        drain_gathers(c, b)
        for d in out_descs(c - 2, b):
            d.wait()
        transpose_add(c, b)
        for d in out_descs(c, b):
            d.start()

    for b in range(2):
        for d in out_descs(_N_CHUNKS - 2 + b, b):
            d.wait()


_emb = functools.partial(
    pl.kernel,
    out_type=jax.ShapeDtypeStruct((_SEQ_LEN, _EG, _BT, 8, 128), jnp.float32),
    mesh=plsc.VectorSubcoreMesh(core_axis_name="c", subcore_axis_name="s"),
    scratch_types=[
        pltpu.VMEM((_S_PER_CHUNK, _BBLK), jnp.int32),
        pltpu.VMEM((_S_PER_CHUNK, _BBLK), jnp.int32),
        pltpu.VMEM((_CHUNK_ROWS, _EMBED_DIM), jnp.float32),
        pltpu.VMEM((_CHUNK_ROWS, _EMBED_DIM), jnp.float32),
        pltpu.VMEM((_S_PER_CHUNK, _EG, 8, _PITCH), jnp.float32),
        pltpu.VMEM((_S_PER_CHUNK, _EG, 8, _PITCH), jnp.float32),
        pltpu.VMEM((_SEQ_LEN, _EMBED_DIM), jnp.float32),
        pltpu.SemaphoreType.DMA,
        pltpu.SemaphoreType.DMA,
        pltpu.SemaphoreType.DMA,
        pltpu.SemaphoreType.DMA,
    ],
    compiler_params=pltpu.CompilerParams(
        use_tc_tiling_on_sc=False, needs_layout_passes=False
    ),
)(_emb_body)


_TCB = 16384


def _repack_body(t_ref, o_ref):
    o_ref[:, 0:_EMBED_DIM] = t_ref[...].T


_repack = pl.pallas_call(
    _repack_body,
    grid=((_VOCAB + _TCB - 1) // _TCB,),
    in_specs=[pl.BlockSpec((_EMBED_DIM, _TCB), lambda i: (0, i))],
    out_specs=pl.BlockSpec((_TCB, 128), lambda i: (i, 0)),
    out_shape=jax.ShapeDtypeStruct((_VOCAB, 128), jnp.float32),
)


@jax.jit
def kernel(inputs, token_table, position_table):
    idx4d = (
        inputs.astype(jnp.int32).T
        .reshape(_ST, 8, _BT, 128)
        .transpose(0, 2, 1, 3)
    ) * 4
    rm_table = _repack(token_table.T).reshape(4 * _VOCAB, _EMBED_DIM)
    flat = _emb(idx4d, rm_table, position_table)
    out = (
        flat.reshape(_SEQ_LEN, _EG, _BT, 8, 128)
        .transpose(2, 4, 0, 1, 3)
        .reshape(_BATCH, _SEQ_LEN, _EMBED_DIM)
    )
    return out


# speedup vs baseline: 7.8915x; 1.0147x over previous
import functools

import jax
import jax.numpy as jnp
from jax import lax
from jax.experimental import pallas as pl
from jax.experimental.pallas import tpu as pltpu
from jax.experimental.pallas import tpu_sc as plsc

_VOCAB = 1000000
_SEQ_LEN = 200
_EMBED_DIM = 32
_BATCH = 4096

_NC = 2
_NS = 16
_NW = _NC * _NS

_BBLK = _BATCH // _NW
_S_PER_CHUNK = 4
_CHUNK_ROWS = _S_PER_CHUNK * _BBLK
_N_CHUNKS = _SEQ_LEN // _S_PER_CHUNK

_ST = _SEQ_LEN // 8
_BT = _BATCH // 128

_EG = _EMBED_DIM // 8
_TILE = 8 * 128
_PITCH = 129

_HALF = 16


def _emb_body(idx_hbm, tok_hbm, pos_hbm, out_hbm,
              idx0, idx1, g0, g1, t0, t1, pos_v,
              gsem0, gsem1, osem0, osem1):
    idxs = (idx0, idx1)
    gs = (g0, g1)
    tiles = (t0, t1)
    gsems = (gsem0, gsem1)
    osems = (osem0, osem1)

    wid = lax.axis_index("s") * _NC + lax.axis_index("c")

    pltpu.sync_copy(pos_hbm, pos_v)

    lane = lax.iota(jnp.int32, 16)
    gv0 = lane >> 3
    gv1 = gv0 + 2
    e8v = lane & 7

    def idx_and_gathers(c, b):
        pltpu.sync_copy(
            idx_hbm.at[c // 2, wid, pl.ds((c % 2) * _S_PER_CHUNK, _S_PER_CHUNK)],
            idxs[b],
        )
        for d in gather_descs(c, b):
            d.start()

    def gather_descs(c, b):
        return [
            pltpu.make_async_copy(
                tok_hbm.at[idxs[b].at[j]],
                gs[b].at[pl.ds(j * _BBLK, _BBLK)],
                gsems[b],
            )
            for j in range(_S_PER_CHUNK)
        ]

    def drain_gathers(c, b):
        for d in gather_descs(c, b):
            d.wait()

    def transpose_add(c, b):
        for j in range(_S_PER_CHUNK):
            s = c * _S_PER_CHUNK + j
            pv0 = pos_v[s, pl.ds(0, _HALF)]
            pv1 = pos_v[s, pl.ds(_HALF, _HALF)]

            @plsc.parallel_loop(0, _BBLK, unroll=4)
            def _(bl):
                r = j * _BBLK + bl
                wv = lax.broadcast_in_dim(bl, (16,), ())
                v0 = gs[b][r, pl.ds(0, _HALF)] + pv0
                v1 = gs[b][r, pl.ds(_HALF, _HALF)] + pv1
                plsc.store_scatter(tiles[b].at[j], [gv0, e8v, wv], v0)
                plsc.store_scatter(tiles[b].at[j], [gv1, e8v, wv], v1)

    def out_descs(c, b):
        ds_ = []
        for j in range(_S_PER_CHUNK):
            s = c * _S_PER_CHUNK + j
            ds_.append(
                pltpu.make_async_copy(
                    tiles[b].at[j, slice(None), slice(None), pl.ds(0, 128)],
                    out_hbm.at[s, slice(None), wid],
                    osems[b],
                )
            )
        return ds_

    idx_and_gathers(0, 0)
    idx_and_gathers(1, 1)

    for b in range(2):
        drain_gathers(b, b)
        transpose_add(b, b)
        for d in out_descs(b, b):
            d.start()
        idx_and_gathers(b + 2, b)

    @pl.loop(2, _N_CHUNKS - 2, step=2)
    def _(sc):
        for b in range(2):
            c = sc + b
            drain_gathers(c, b)
            for d in out_descs(c - 2, b):
                d.wait()
            transpose_add(c, b)
            for d in out_descs(c, b):
                d.start()
            idx_and_gathers(c + 2, b)

    for b in range(2):
        c = _N_CHUNKS - 2 + b
        drain_gathers(c, b)
        for d in out_descs(c - 2, b):
            d.wait()
        transpose_add(c, b)
        for d in out_descs(c, b):
            d.start()

    for b in range(2):
        for d in out_descs(_N_CHUNKS - 2 + b, b):
            d.wait()


_emb = functools.partial(
    pl.kernel,
    out_type=jax.ShapeDtypeStruct((_SEQ_LEN, _EG, _BT, 8, 128), jnp.float32),
    mesh=plsc.VectorSubcoreMesh(core_axis_name="c", subcore_axis_name="s"),
    scratch_types=[
        pltpu.VMEM((_S_PER_CHUNK, _BBLK), jnp.int32),
        pltpu.VMEM((_S_PER_CHUNK, _BBLK), jnp.int32),
        pltpu.VMEM((_CHUNK_ROWS, _EMBED_DIM), jnp.float32),
        pltpu.VMEM((_CHUNK_ROWS, _EMBED_DIM), jnp.float32),
        pltpu.VMEM((_S_PER_CHUNK, _EG, 8, _PITCH), jnp.float32),
        pltpu.VMEM((_S_PER_CHUNK, _EG, 8, _PITCH), jnp.float32),
        pltpu.VMEM((_SEQ_LEN, _EMBED_DIM), jnp.float32),
        pltpu.SemaphoreType.DMA,
        pltpu.SemaphoreType.DMA,
        pltpu.SemaphoreType.DMA,
        pltpu.SemaphoreType.DMA,
    ],
    compiler_params=pltpu.CompilerParams(
        use_tc_tiling_on_sc=False, needs_layout_passes=False
    ),
)(_emb_body)


_TCB = 32768


def _repack_body(t_ref, o_ref):
    o_ref[:, 0:_EMBED_DIM] = t_ref[...].T


_repack = pl.pallas_call(
    _repack_body,
    grid=((_VOCAB + _TCB - 1) // _TCB,),
    in_specs=[pl.BlockSpec((_EMBED_DIM, _TCB), lambda i: (0, i))],
    out_specs=pl.BlockSpec((_TCB, 128), lambda i: (i, 0)),
    out_shape=jax.ShapeDtypeStruct((_VOCAB, 128), jnp.float32),
)


@jax.jit
def kernel(inputs, token_table, position_table):
    idx4d = (
        inputs.astype(jnp.int32).T
        .reshape(_ST, 8, _BT, 128)
        .transpose(0, 2, 1, 3)
    ) * 4
    rm_table = _repack(token_table.T).reshape(4 * _VOCAB, _EMBED_DIM)
    flat = _emb(idx4d, rm_table, position_table)
    out = (
        flat.reshape(_SEQ_LEN, _EG, _BT, 8, 128)
        .transpose(2, 4, 0, 1, 3)
        .reshape(_BATCH, _SEQ_LEN, _EMBED_DIM)
    )
    return out
